# baseline (speedup 1.0000x reference)
import functools

import jax
import jax.numpy as jnp
from jax import lax
from jax.experimental import pallas as pl
from jax.experimental.pallas import tpu as pltpu
from jax.experimental.pallas import tpu_sc as plsc

NUM_LAYERS = 3
K = 2048
D = 256
T = 512
B = 16

CHUNKS = 4
BC = B // CHUNKS
TOK_C = BC * T

NC = 2
NS = 16
NW = NC * NS
ROWS_PER_W = TOK_C // NW
assert ROWS_PER_W <= 128


def _dist_body(n_prev, h_ref, *rest):
    q_refs = rest[:n_prev]
    w_ref, idx_ref = rest[n_prev], rest[n_prev + 1]
    res = h_ref[0]
    for q_ref in q_refs:
        res = res - q_ref[0]
    w = w_ref[...]
    r2 = jnp.sum(res * res, axis=1, keepdims=True)
    w2 = jnp.sum(w * w, axis=1)
    cross = lax.dot_general(
        res, w, (((1,), (1,)), ((), ())),
        precision=lax.Precision.DEFAULT)
    dists = r2 - 2.0 * cross + w2[None, :]
    m = jnp.min(dists, axis=1, keepdims=True)
    iota = lax.broadcasted_iota(jnp.int32, dists.shape, 1)
    idx_ref[0, 0, :] = jnp.min(jnp.where(dists == m, iota, K), axis=1)


def _dist_call(n_prev, h, qs, W):
    row_spec = pl.BlockSpec((1, T, D), lambda b: (b, 0, 0))
    return pl.pallas_call(
        functools.partial(_dist_body, n_prev),
        grid=(BC,),
        in_specs=[row_spec] * (1 + n_prev) + [pl.BlockSpec((K, D), lambda b: (0, 0))],
        out_specs=pl.BlockSpec((1, 1, T), lambda b: (b, 0, 0)),
        out_shape=jax.ShapeDtypeStruct((BC, 1, T), jnp.int32),
    )(h, *qs, W)


def _gather_body(table_hbm, idx_hbm, out_hbm, idx_v, rows_v, sem):
    wid = lax.axis_index("s") * NC + lax.axis_index("c")
    pltpu.sync_copy(idx_hbm.at[wid], idx_v)
    pltpu.async_copy(table_hbm.at[idx_v], rows_v, sem).wait()
    pltpu.sync_copy(rows_v, out_hbm.at[pl.ds(wid * ROWS_PER_W, ROWS_PER_W)])


_sc_gather = pl.kernel(
    _gather_body,
    out_type=jax.ShapeDtypeStruct((TOK_C, D), jnp.float32),
    mesh=plsc.VectorSubcoreMesh(core_axis_name="c", subcore_axis_name="s"),
    scratch_types=[
        pltpu.VMEM((ROWS_PER_W,), jnp.int32),
        pltpu.VMEM((ROWS_PER_W, D), jnp.float32),
        pltpu.SemaphoreType.DMA,
    ],
)


def _add3_body(a_ref, b_ref, c_ref, o_ref):
    o_ref[...] = (a_ref[...] + b_ref[...]) + c_ref[...]


def _add3(a, b, c):
    row_spec = pl.BlockSpec((1, T, D), lambda i: (i, 0, 0))
    return pl.pallas_call(
        _add3_body,
        grid=(BC,),
        in_specs=[row_spec] * 3,
        out_specs=row_spec,
        out_shape=jax.ShapeDtypeStruct((BC, T, D), jnp.float32),
    )(a, b, c)


@jax.jit
def kernel(h, W0, W1, W2):
    ws = (W0, W1, W2)
    hs = [h[c * BC:(c + 1) * BC] for c in range(CHUNKS)]
    idx = [[None] * CHUNKS for _ in range(NUM_LAYERS)]
    q = [[None] * CHUNKS for _ in range(NUM_LAYERS)]
    for l in range(NUM_LAYERS):
        for c in range(CHUNKS):
            idx[l][c] = _dist_call(l, hs[c], tuple(q[m][c] for m in range(l)),
                                   ws[l])
            q[l][c] = _sc_gather(
                ws[l], idx[l][c].reshape(NW, ROWS_PER_W)).reshape(BC, T, D)
    qt = jnp.concatenate(
        [_add3(q[0][c], q[1][c], q[2][c]) for c in range(CHUNKS)], axis=0)
    outs = [jnp.concatenate([idx[l][c].reshape(BC, T) for c in range(CHUNKS)],
                            axis=0) for l in range(NUM_LAYERS)]
    return (qt, outs[0], outs[1], outs[2])


# speedup vs baseline: 1.2644x; 1.2644x over previous
import functools

import jax
import jax.numpy as jnp
from jax import lax
from jax.experimental import pallas as pl
from jax.experimental.pallas import tpu as pltpu
from jax.experimental.pallas import tpu_sc as plsc

NUM_LAYERS = 3
K = 2048
D = 256
T = 512
B = 16

CHUNKS = 2
BC = B // CHUNKS
TOK_C = BC * T

NC = 2
NS = 16
NW = NC * NS
ROWS_PER_W = TOK_C // NW
assert ROWS_PER_W <= 128


def _argmin_scan(r2, cross, w2):
    G = K // 128
    iota128 = lax.broadcasted_iota(jnp.int32, (T, 128), 1)
    run_min = None
    for g in range(G):
        dg = r2[:, 0:1] - 2.0 * cross[:, g * 128:(g + 1) * 128] \
            + w2[None, g * 128:(g + 1) * 128]
        if run_min is None:
            run_min, run_idx = dg, iota128
        else:
            lt = dg < run_min
            run_min = jnp.where(lt, dg, run_min)
            run_idx = jnp.where(lt, iota128 + g * 128, run_idx)
    m = jnp.min(run_min, axis=1, keepdims=True)
    return jnp.min(jnp.where(run_min == m, run_idx, K), axis=1)


def _dist_body(n_prev, h_ref, *rest):
    q_refs = rest[:n_prev]
    w_ref, idx_ref = rest[n_prev], rest[n_prev + 1]
    res = h_ref[0]
    for q_ref in q_refs:
        res = res - q_ref[0]
    w = w_ref[...]
    r2 = jnp.sum(res * res, axis=1, keepdims=True)
    w2 = jnp.sum(w * w, axis=1)
    cross = lax.dot_general(
        res, w, (((1,), (1,)), ((), ())),
        precision=lax.Precision.DEFAULT)
    idx_ref[0, 0, :] = _argmin_scan(r2, cross, w2)
    if n_prev == 2:
        rest[n_prev + 2][0] = q_refs[0][0] + q_refs[1][0]


def _dist_call(n_prev, h, qs, W):
    row_spec = pl.BlockSpec((1, T, D), lambda b: (b, 0, 0))
    idx_spec = pl.BlockSpec((1, 1, T), lambda b: (b, 0, 0))
    idx_shape = jax.ShapeDtypeStruct((BC, 1, T), jnp.int32)
    if n_prev == 2:
        out_specs = (idx_spec, row_spec)
        out_shape = (idx_shape, jax.ShapeDtypeStruct((BC, T, D), jnp.float32))
    else:
        out_specs = idx_spec
        out_shape = idx_shape
    return pl.pallas_call(
        functools.partial(_dist_body, n_prev),
        grid=(BC,),
        in_specs=[row_spec] * (1 + n_prev) + [pl.BlockSpec((K, D), lambda b: (0, 0))],
        out_specs=out_specs,
        out_shape=out_shape,
    )(h, *qs, W)


def _gather_body(table_hbm, idx_hbm, out_hbm, idx_v, rows_v, sem):
    wid = lax.axis_index("s") * NC + lax.axis_index("c")
    pltpu.sync_copy(idx_hbm.at[wid], idx_v)
    pltpu.async_copy(table_hbm.at[idx_v], rows_v, sem).wait()
    pltpu.sync_copy(rows_v, out_hbm.at[pl.ds(wid * ROWS_PER_W, ROWS_PER_W)])


_sc_gather = pl.kernel(
    _gather_body,
    out_type=jax.ShapeDtypeStruct((TOK_C, D), jnp.float32),
    mesh=plsc.VectorSubcoreMesh(core_axis_name="c", subcore_axis_name="s"),
    scratch_types=[
        pltpu.VMEM((ROWS_PER_W,), jnp.int32),
        pltpu.VMEM((ROWS_PER_W, D), jnp.float32),
        pltpu.SemaphoreType.DMA,
    ],
)


def _gather_add_body(table_hbm, idx_hbm, s01_hbm, out_hbm,
                     idx_v, rows_v, s01_v, sem):
    wid = lax.axis_index("s") * NC + lax.axis_index("c")
    base = wid * ROWS_PER_W
    pltpu.sync_copy(idx_hbm.at[wid], idx_v)
    cp = pltpu.async_copy(table_hbm.at[idx_v], rows_v, sem)
    pltpu.sync_copy(s01_hbm.at[pl.ds(base, ROWS_PER_W)], s01_v)
    cp.wait()

    def row_add(i, carry):
        for c in range(D // 16):
            sl = pl.ds(c * 16, 16)
            rows_v[i, sl] = s01_v[i, sl] + rows_v[i, sl]
        return carry

    lax.fori_loop(0, ROWS_PER_W, row_add, 0)
    pltpu.sync_copy(rows_v, out_hbm.at[pl.ds(base, ROWS_PER_W)])


_sc_gather_add = pl.kernel(
    _gather_add_body,
    out_type=jax.ShapeDtypeStruct((TOK_C, D), jnp.float32),
    mesh=plsc.VectorSubcoreMesh(core_axis_name="c", subcore_axis_name="s"),
    scratch_types=[
        pltpu.VMEM((ROWS_PER_W,), jnp.int32),
        pltpu.VMEM((ROWS_PER_W, D), jnp.float32),
        pltpu.VMEM((ROWS_PER_W, D), jnp.float32),
        pltpu.SemaphoreType.DMA,
    ],
)


@jax.jit
def kernel(h, W0, W1, W2):
    ws = (W0, W1, W2)
    hs = [h[c * BC:(c + 1) * BC] for c in range(CHUNKS)]
    idx = [[None] * CHUNKS for _ in range(NUM_LAYERS)]
    q = [[None] * CHUNKS for _ in range(NUM_LAYERS)]
    qt = [None] * CHUNKS
    for c in range(CHUNKS):
        idx[0][c] = _dist_call(0, hs[c], (), ws[0])
        q[0][c] = _sc_gather(
            ws[0], idx[0][c].reshape(NW, ROWS_PER_W)).reshape(BC, T, D)
    for c in range(CHUNKS):
        idx[1][c] = _dist_call(1, hs[c], (q[0][c],), ws[1])
        q[1][c] = _sc_gather(
            ws[1], idx[1][c].reshape(NW, ROWS_PER_W)).reshape(BC, T, D)
    for c in range(CHUNKS):
        idx[2][c], s01 = _dist_call(2, hs[c], (q[0][c], q[1][c]), ws[2])
        qt[c] = _sc_gather_add(
            ws[2], idx[2][c].reshape(NW, ROWS_PER_W),
            s01.reshape(TOK_C, D)).reshape(BC, T, D)
    outs = [jnp.concatenate([idx[l][c].reshape(BC, T) for c in range(CHUNKS)],
                            axis=0) for l in range(NUM_LAYERS)]
    return (jnp.concatenate(qt, axis=0), outs[0], outs[1], outs[2])


# speedup vs baseline: 1.3105x; 1.0365x over previous
import functools

import jax
import jax.numpy as jnp
from jax import lax
from jax.experimental import pallas as pl
from jax.experimental.pallas import tpu as pltpu
from jax.experimental.pallas import tpu_sc as plsc

NUM_LAYERS = 3
K = 2048
D = 256
T = 512
B = 16

CHUNKS = 2
BC = B // CHUNKS
TOK_C = BC * T

NC = 2
NS = 16
NW = NC * NS
ROWS_PER_W = TOK_C // NW
assert ROWS_PER_W <= 128


def _argmin_scan(r2, cross, w2):
    rows = cross.shape[0]
    G = K // 128
    iota128 = lax.broadcasted_iota(jnp.int32, (rows, 128), 1)
    run_min = None
    for g in range(G):
        dg = r2[:, 0:1] - 2.0 * cross[:, g * 128:(g + 1) * 128] \
            + w2[None, g * 128:(g + 1) * 128]
        if run_min is None:
            run_min, run_idx = dg, iota128
        else:
            lt = dg < run_min
            run_min = jnp.where(lt, dg, run_min)
            run_idx = jnp.where(lt, iota128 + g * 128, run_idx)
    m = jnp.min(run_min, axis=1, keepdims=True)
    return jnp.min(jnp.where(run_min == m, run_idx, K), axis=1)


HALVES = 2
TT = T // HALVES


def _dist_body(n_prev, h_ref, *rest):
    q_refs = rest[:n_prev]
    w_ref, idx_ref = rest[n_prev], rest[n_prev + 1]
    res = h_ref[0]
    for q_ref in q_refs:
        res = res - q_ref[0]
    w = w_ref[...]
    w2 = jnp.sum(w * w, axis=1)
    for hi in range(HALVES):
        res_h = res[hi * TT:(hi + 1) * TT]
        r2 = jnp.sum(res_h * res_h, axis=1, keepdims=True)
        cross = lax.dot_general(
            res_h, w, (((1,), (1,)), ((), ())),
            precision=lax.Precision.DEFAULT)
        idx_ref[0, 0, hi * TT:(hi + 1) * TT] = _argmin_scan(r2, cross, w2)
    if n_prev == 2:
        rest[n_prev + 2][0] = q_refs[0][0] + q_refs[1][0]


def _dist_call(n_prev, h, qs, W):
    row_spec = pl.BlockSpec((1, T, D), lambda b: (b, 0, 0))
    idx_spec = pl.BlockSpec((1, 1, T), lambda b: (b, 0, 0))
    idx_shape = jax.ShapeDtypeStruct((BC, 1, T), jnp.int32)
    if n_prev == 2:
        out_specs = (idx_spec, row_spec)
        out_shape = (idx_shape, jax.ShapeDtypeStruct((BC, T, D), jnp.float32))
    else:
        out_specs = idx_spec
        out_shape = idx_shape
    return pl.pallas_call(
        functools.partial(_dist_body, n_prev),
        grid=(BC,),
        in_specs=[row_spec] * (1 + n_prev) + [pl.BlockSpec((K, D), lambda b: (0, 0))],
        out_specs=out_specs,
        out_shape=out_shape,
    )(h, *qs, W)


def _gather_body(table_hbm, idx_hbm, out_hbm, idx_v, rows_v, sem):
    wid = lax.axis_index("s") * NC + lax.axis_index("c")
    pltpu.sync_copy(idx_hbm.at[wid], idx_v)
    pltpu.async_copy(table_hbm.at[idx_v], rows_v, sem).wait()
    pltpu.sync_copy(rows_v, out_hbm.at[pl.ds(wid * ROWS_PER_W, ROWS_PER_W)])


_sc_gather = pl.kernel(
    _gather_body,
    out_type=jax.ShapeDtypeStruct((TOK_C, D), jnp.float32),
    mesh=plsc.VectorSubcoreMesh(core_axis_name="c", subcore_axis_name="s"),
    scratch_types=[
        pltpu.VMEM((ROWS_PER_W,), jnp.int32),
        pltpu.VMEM((ROWS_PER_W, D), jnp.float32),
        pltpu.SemaphoreType.DMA,
    ],
)


def _gather_add_body(table_hbm, idx_hbm, s01_hbm, out_hbm,
                     idx_v, rows_v, s01_v, sem):
    wid = lax.axis_index("s") * NC + lax.axis_index("c")
    base = wid * ROWS_PER_W
    pltpu.sync_copy(idx_hbm.at[wid], idx_v)
    cp = pltpu.async_copy(table_hbm.at[idx_v], rows_v, sem)
    pltpu.sync_copy(s01_hbm.at[pl.ds(base, ROWS_PER_W)], s01_v)
    cp.wait()

    def row_add(i, carry):
        for c in range(D // 16):
            sl = pl.ds(c * 16, 16)
            rows_v[i, sl] = s01_v[i, sl] + rows_v[i, sl]
        return carry

    lax.fori_loop(0, ROWS_PER_W, row_add, 0)
    pltpu.sync_copy(rows_v, out_hbm.at[pl.ds(base, ROWS_PER_W)])


_sc_gather_add = pl.kernel(
    _gather_add_body,
    out_type=jax.ShapeDtypeStruct((TOK_C, D), jnp.float32),
    mesh=plsc.VectorSubcoreMesh(core_axis_name="c", subcore_axis_name="s"),
    scratch_types=[
        pltpu.VMEM((ROWS_PER_W,), jnp.int32),
        pltpu.VMEM((ROWS_PER_W, D), jnp.float32),
        pltpu.VMEM((ROWS_PER_W, D), jnp.float32),
        pltpu.SemaphoreType.DMA,
    ],
)


@jax.jit
def kernel(h, W0, W1, W2):
    ws = (W0, W1, W2)
    hs = [h[c * BC:(c + 1) * BC] for c in range(CHUNKS)]
    idx = [[None] * CHUNKS for _ in range(NUM_LAYERS)]
    q = [[None] * CHUNKS for _ in range(NUM_LAYERS)]
    qt = [None] * CHUNKS
    for c in range(CHUNKS):
        idx[0][c] = _dist_call(0, hs[c], (), ws[0])
        q[0][c] = _sc_gather(
            ws[0], idx[0][c].reshape(NW, ROWS_PER_W)).reshape(BC, T, D)
    for c in range(CHUNKS):
        idx[1][c] = _dist_call(1, hs[c], (q[0][c],), ws[1])
        q[1][c] = _sc_gather(
            ws[1], idx[1][c].reshape(NW, ROWS_PER_W)).reshape(BC, T, D)
    for c in range(CHUNKS):
        idx[2][c], s01 = _dist_call(2, hs[c], (q[0][c], q[1][c]), ws[2])
        qt[c] = _sc_gather_add(
            ws[2], idx[2][c].reshape(NW, ROWS_PER_W),
            s01.reshape(TOK_C, D)).reshape(BC, T, D)
    outs = [jnp.concatenate([idx[l][c].reshape(BC, T) for c in range(CHUNKS)],
                            axis=0) for l in range(NUM_LAYERS)]
    return (jnp.concatenate(qt, axis=0), outs[0], outs[1], outs[2])
